# baseline (speedup 1.0000x reference)
import functools

import numpy as np
import jax
import jax.numpy as jnp
from jax import lax
from jax.experimental import pallas as pl
from jax.experimental.pallas import tpu as pltpu
from jax.experimental.pallas import tpu_sc as plsc

_H = 2048
_W = 2048
_CH = 4
_NWORK = 32
_RPW = _H // _NWORK
_SEG = 128
_NCHUNKB = 96
_STRIPE = _NCHUNKB * 16

_NORM = np.sqrt(np.float32(4.0 * 4.0 + 1024.0 * 1024.0 + 1024.0 * 1024.0),
                dtype=np.float32)
_S = np.float32(np.float32(_NORM) * np.float32(0.5) / np.float32(2048.0))


def _prep_kernel(cos_ref, sin_ref, nv_ref):
    t = lax.broadcasted_iota(jnp.int32, (_H, 16), 0).astype(jnp.float32)
    ang = t * 2.0 * np.float32(np.pi) / 2048.0
    c = jnp.cos(ang)
    s = jnp.sin(ang)
    cos_ref[...] = c
    sin_ref[...] = s
    asc = jnp.abs(c) * _S
    ass = jnp.abs(s) * _S
    bx = jnp.where(asc > 0.0, 512.0 / jnp.maximum(asc, 1e-30), 3000.0)
    by = jnp.where(ass > 0.0, 2.0 / jnp.maximum(ass, 1e-30), 3000.0)
    bound = jnp.minimum(jnp.minimum(bx, by), 3000.0)
    nv_ref[...] = jnp.clip((bound * (1.0 / 16.0)).astype(jnp.int32) + 3,
                           1, _NCHUNKB)


_prep = pl.pallas_call(
    _prep_kernel,
    out_shape=(jax.ShapeDtypeStruct((_H, 16), jnp.float32),
               jax.ShapeDtypeStruct((_H, 16), jnp.float32),
               jax.ShapeDtypeStruct((_H, 16), jnp.int32)),
)


def _zero_kernel(o_ref):
    o_ref[...] = jnp.zeros_like(o_ref)


_zeros = pl.pallas_call(
    _zero_kernel,
    out_shape=jax.ShapeDtypeStruct((_CH, _H, _W), jnp.float32),
    grid=(16,),
    out_specs=pl.BlockSpec((_CH, _H // 16, _W), lambda i: (0, i, 0)),
)

_mesh = plsc.VectorSubcoreMesh(core_axis_name="c", subcore_axis_name="s")


@functools.partial(
    pl.kernel,
    mesh=_mesh,
    out_type=(),
    scratch_types=[
        pltpu.VMEM((_CH * 4 * 1024,), jnp.float32),
        pltpu.VMEM((_RPW, 16), jnp.float32),
        pltpu.VMEM((_RPW, 16), jnp.float32),
        pltpu.VMEM((_RPW, 16), jnp.int32),
        pltpu.VMEM((2, _CH, 1, _STRIPE), jnp.float32),
        pltpu.SemaphoreType.DMA,
        pltpu.SemaphoreType.DMA,
    ],
    compiler_params=pltpu.CompilerParams(needs_layout_passes=False),
)
def _remap(tbl_hbm, cosb_hbm, sinb_hbm, nvb_hbm, out_ref,
           table_v, cos_v, sin_v, nv_v, buf_v, sem0, sem1):
    wid = lax.axis_index("s") * 2 + lax.axis_index("c")
    base = wid * _RPW
    pltpu.sync_copy(tbl_hbm, table_v)
    pltpu.sync_copy(cosb_hbm.at[pl.ds(base, _RPW)], cos_v)
    pltpu.sync_copy(sinb_hbm.at[pl.ds(base, _RPW)], sin_v)
    pltpu.sync_copy(nvb_hbm.at[pl.ds(base, _RPW)], nv_v)
    iota16 = lax.iota(jnp.int32, 16)
    zeros16 = jnp.zeros((16,), jnp.float32)
    sems = (sem0, sem1)

    def drain(b, n):
        def dwait(j, c2):
            pltpu.make_async_copy(
                out_ref.at[:, pl.ds(0, 1), pl.ds(0, _SEG)],
                buf_v.at[b, :, :, pl.ds(0, _SEG)],
                sems[b]).wait()
            return c2
        lax.fori_loop(0, n, dwait, 0)

    def group(g, carry):
        nvp0, nsp0, nvp1, nsp1 = carry
        prev = ((nvp0, nsp0), (nvp1, nsp1))
        new = []
        for b in range(2):
            rl = 2 * g + b
            row = base + rl
            nv_prev, ns_prev = prev[b]
            drain(b, ns_prev)

            def zchunk(k, c2):
                for c in range(_CH):
                    buf_v[b, c, 0, pl.ds(k * 16, 16)] = zeros16
                return c2

            lax.fori_loop(0, nv_prev, zchunk, 0)

            cv = cos_v[rl]
            sv = sin_v[rl]
            nv = jnp.max(nv_v[rl])
            nseg = lax.shift_right_logical(nv + 7, 3)

            def chunk(k, c2):
                rrv = (iota16 + k * 16).astype(jnp.float32)
                rho = rrv * _S
                x = 512.0 + rho * cv
                y = 2.0 - rho * sv
                m = (x >= 0.0) & (x < 1024.0) & (y >= 0.0) & (y < 4.0)
                xi = jnp.clip(x.astype(jnp.int32), 0, 1023)
                yi = jnp.clip(y.astype(jnp.int32), 0, 3)
                idx = yi * 1024 + xi
                for c in range(_CH):
                    val = plsc.load_gather(table_v, [idx + (c * 4096)])
                    buf_v[b, c, 0, pl.ds(k * 16, 16)] = jnp.where(m, val, 0.0)
                return c2

            lax.fori_loop(0, nv, chunk, 0)

            def seg(j, c2):
                pltpu.async_copy(
                    buf_v.at[b, :, :, pl.ds(j * _SEG, _SEG)],
                    out_ref.at[:, pl.ds(row, 1), pl.ds(j * _SEG, _SEG)],
                    sems[b])
                return c2

            lax.fori_loop(0, nseg, seg, 0)
            new.append((nv, nseg))
        return (new[0][0], new[0][1], new[1][0], new[1][1])

    fin = lax.fori_loop(0, _RPW // 2, group,
                        (jnp.int32(_NCHUNKB), jnp.int32(0),
                         jnp.int32(_NCHUNKB), jnp.int32(0)))
    drain(0, fin[1])
    drain(1, fin[3])


def kernel(data):
    cos_b, sin_b, nv_b = _prep()
    tbl = data[:, :4, :].reshape(_CH * 4 * 1024)
    z = _zeros()
    zref = jax.new_ref(z)
    _remap(tbl, cos_b, sin_b, nv_b, zref)
    return zref[...]
